# baseline (speedup 1.0000x reference)
import functools

import jax
import jax.numpy as jnp
from jax import lax
from jax.experimental import pallas as pl
from jax.experimental.pallas import tpu as pltpu
from jax.experimental.pallas import tpu_sc as plsc

N = 10000
NP = 10240
E = 320000
EP = 327680
NC, NS, L = 2, 16, 16
NW = NC * NS
EW = EP // NW
NBLK = EW // L

_MESH = plsc.VectorSubcoreMesh(
    core_axis_name="c", subcore_axis_name="s", num_cores=NC, num_subcores=NS)
_SC_PARAMS = pltpu.CompilerParams(needs_layout_passes=False)


def _wid():
    return lax.axis_index("s") * NC + lax.axis_index("c")


def _make_sc_pass1(n8):
    tw = n8 * NP * 2

    def body(xl8_hbm, xr8_hbm, src_hbm, dst_hbm, attb_hbm,
             p_hbm, s1_hbm,
             src_v, dst_v, lg_v, s1_v, attb_v, xl_v0, xl_v1, xr_v0, xr_v1,
             sem_l, sem_r):
        w = _wid()
        base = w * EW
        xlb = (xl_v0, xl_v1)
        xrb = (xr_v0, xr_v1)
        cpl = pltpu.async_copy(xl8_hbm.at[pl.ds(0, NP * 2)], xl_v0, sem_l)
        cpr = pltpu.async_copy(xr8_hbm.at[pl.ds(0, NP * 2)], xr_v0, sem_r)
        pltpu.sync_copy(src_hbm.at[pl.ds(base, EW)], src_v)
        pltpu.sync_copy(dst_hbm.at[pl.ds(base, EW)], dst_v)
        pltpu.sync_copy(attb_hbm, attb_v)

        @plsc.parallel_loop(0, NBLK, unroll=8)
        def _(b):
            lg_v[pl.ds(b * L, L)] = jnp.zeros((L,), jnp.float32)

        @plsc.parallel_loop(0, NP // L, unroll=4)
        def _(b):
            s1_v[pl.ds(b * L, L)] = jnp.zeros((L,), jnp.float32)

        for e in range(n8):
            xl_v = xlb[e % 2]
            xr_v = xrb[e % 2]
            cpl.wait()
            cpr.wait()
            if e + 1 < n8:
                cpl = pltpu.async_copy(
                    xl8_hbm.at[pl.ds((e + 1) * NP * 2, NP * 2)],
                    xlb[(e + 1) % 2], sem_l)
                cpr = pltpu.async_copy(
                    xr8_hbm.at[pl.ds((e + 1) * NP * 2, NP * 2)],
                    xrb[(e + 1) % 2], sem_r)
            att0 = attb_v[pl.ds((e * 2 + 0) * L, L)]
            att1 = attb_v[pl.ds((e * 2 + 1) * L, L)]

            @plsc.parallel_loop(0, NBLK, unroll=8)
            def _(b, att0=att0, att1=att1):
                s16 = src_v[pl.ds(b * L, L)]
                d16 = dst_v[pl.ds(b * L, L)]
                acc = lg_v[pl.ds(b * L, L)]
                z0 = (plsc.load_gather(xl_v, [s16]) +
                      plsc.load_gather(xr_v, [d16]))
                z0 = jnp.maximum(z0, 0.2 * z0)
                acc = acc + z0 * att0
                z1 = (plsc.load_gather(xl_v, [s16 + NP]) +
                      plsc.load_gather(xr_v, [d16 + NP]))
                z1 = jnp.maximum(z1, 0.2 * z1)
                acc = acc + z1 * att1
                lg_v[pl.ds(b * L, L)] = acc

        @plsc.parallel_loop(0, NBLK, unroll=8)
        def _(b):
            p16 = jnp.exp(lg_v[pl.ds(b * L, L)])
            lg_v[pl.ds(b * L, L)] = p16
            d16 = dst_v[pl.ds(b * L, L)]
            plsc.addupdate_scatter(s1_v, [d16], p16)

        pltpu.sync_copy(lg_v, p_hbm.at[pl.ds(base, EW)])
        pltpu.sync_copy(s1_v, s1_hbm.at[pl.ds(w * NP, NP)])

    return pl.kernel(
        body,
        out_type=(jax.ShapeDtypeStruct((EP,), jnp.float32),
                  jax.ShapeDtypeStruct((NW * NP,), jnp.float32)),
        mesh=_MESH,
        compiler_params=_SC_PARAMS,
        scratch_types=[
            pltpu.VMEM((EW,), jnp.int32),
            pltpu.VMEM((EW,), jnp.int32),
            pltpu.VMEM((EW,), jnp.float32),
            pltpu.VMEM((NP,), jnp.float32),
            pltpu.VMEM((16 * L,), jnp.float32),
            pltpu.VMEM((NP * 2,), jnp.float32),
            pltpu.VMEM((NP * 2,), jnp.float32),
            pltpu.VMEM((NP * 2,), jnp.float32),
            pltpu.VMEM((NP * 2,), jnp.float32),
            pltpu.SemaphoreType.DMA,
            pltpu.SemaphoreType.DMA,
        ],
    )


def _make_sc_pass2(n8):
    ow = NP * 2

    def body(xl8_hbm, src_hbm, dst_hbm, p_hbm, s1t_hbm, tb_hbm,
             num_hbm, den_hbm,
             src_v, dst_v, al_v, s1t_v, tb_v, xl_v0, xl_v1, num_v, den_v,
             sem_l):
        w = _wid()
        base = w * EW
        xlb = (xl_v0, xl_v1)
        cpl = pltpu.async_copy(xl8_hbm.at[pl.ds(0, NP * 2)], xl_v0, sem_l)
        pltpu.sync_copy(src_hbm.at[pl.ds(base, EW)], src_v)
        pltpu.sync_copy(dst_hbm.at[pl.ds(base, EW)], dst_v)
        pltpu.sync_copy(p_hbm.at[pl.ds(base, EW)], al_v)
        pltpu.sync_copy(s1t_hbm, s1t_v)
        pltpu.sync_copy(tb_hbm, tb_v)
        tv = tb_v[...]

        @plsc.parallel_loop(0, NBLK, unroll=8)
        def _(b):
            d16 = dst_v[pl.ds(b * L, L)]
            sg = plsc.load_gather(s1t_v, [d16])
            al_v[pl.ds(b * L, L)] = (al_v[pl.ds(b * L, L)] /
                                     (sg + jnp.float32(1e-16)))

        for e in range(n8):
            xl_v = xlb[e % 2]
            cpl.wait()
            if e + 1 < n8:
                cpl = pltpu.async_copy(
                    xl8_hbm.at[pl.ds((e + 1) * NP * 2, NP * 2)],
                    xlb[(e + 1) % 2], sem_l)

            @plsc.parallel_loop(0, ow // L, unroll=4)
            def _(b):
                num_v[pl.ds(b * L, L)] = jnp.zeros((L,), jnp.float32)
                den_v[pl.ds(b * L, L)] = jnp.zeros((L,), jnp.float32)

            @plsc.parallel_loop(0, NBLK, unroll=8)
            def _(b):
                s16 = src_v[pl.ds(b * L, L)]
                d16 = dst_v[pl.ds(b * L, L)]
                a16 = al_v[pl.ds(b * L, L)]
                m0 = plsc.load_gather(xl_v, [s16]) * a16
                q0 = jnp.exp(m0 * tv)
                plsc.addupdate_scatter(den_v, [d16], q0)
                plsc.addupdate_scatter(num_v, [d16], q0 * m0)
                m1 = plsc.load_gather(xl_v, [s16 + NP]) * a16
                q1 = jnp.exp(m1 * tv)
                plsc.addupdate_scatter(den_v, [d16 + NP], q1)
                plsc.addupdate_scatter(num_v, [d16 + NP], q1 * m1)

            off = (w * n8 + e) * ow
            pltpu.sync_copy(num_v, num_hbm.at[pl.ds(off, ow)])
            pltpu.sync_copy(den_v, den_hbm.at[pl.ds(off, ow)])

    return pl.kernel(
        body,
        out_type=(jax.ShapeDtypeStruct((NW * n8 * ow,), jnp.float32),
                  jax.ShapeDtypeStruct((NW * n8 * ow,), jnp.float32)),
        mesh=_MESH,
        compiler_params=_SC_PARAMS,
        scratch_types=[
            pltpu.VMEM((EW,), jnp.int32),
            pltpu.VMEM((EW,), jnp.int32),
            pltpu.VMEM((EW,), jnp.float32),
            pltpu.VMEM((NP,), jnp.float32),
            pltpu.VMEM((L,), jnp.float32),
            pltpu.VMEM((NP * 2,), jnp.float32),
            pltpu.VMEM((NP * 2,), jnp.float32),
            pltpu.VMEM((NP * 2,), jnp.float32),
            pltpu.VMEM((NP * 2,), jnp.float32),
            pltpu.SemaphoreType.DMA,
        ],
    )


_SC_P1_16 = _make_sc_pass1(8)
_SC_P2_16 = _make_sc_pass2(8)
_SC_P1_8 = _make_sc_pass1(4)
_SC_P2_8 = _make_sc_pass2(4)


def _mm_body(x_ref, w_ref, b_ref, o_ref):
    o_ref[...] = (jnp.dot(x_ref[...], w_ref[...],
                          preferred_element_type=jnp.float32) + b_ref[...])


def _tc_matmul(x, w, b):
    n, d = x.shape
    k = w.shape[1]
    blk = 2048
    return pl.pallas_call(
        _mm_body,
        out_shape=jax.ShapeDtypeStruct((n, k), jnp.float32),
        grid=(n // blk,),
        in_specs=[pl.BlockSpec((blk, d), lambda i: (i, 0)),
                  pl.BlockSpec((d, k), lambda i: (0, 0)),
                  pl.BlockSpec((1, k), lambda i: (0, 0))],
        out_specs=pl.BlockSpec((blk, k), lambda i: (i, 0)),
    )(x, w, b.reshape(1, k))


def _red_body(p_ref, o_ref):
    o_ref[...] = jnp.sum(p_ref[...], axis=0)


def _tc_reduce(parts, rows, cols):
    blk = 8192 if cols % 8192 == 0 else 2048
    return pl.pallas_call(
        _red_body,
        out_shape=jax.ShapeDtypeStruct((cols,), jnp.float32),
        grid=(cols // blk,),
        in_specs=[pl.BlockSpec((rows, blk), lambda i: (0, i))],
        out_specs=pl.BlockSpec((blk,), lambda i: (i,)),
    )(parts.reshape(rows, cols))


def _red2_body(a_ref, b_ref, oa_ref, ob_ref):
    oa_ref[...] = jnp.sum(a_ref[...], axis=0)
    ob_ref[...] = jnp.sum(b_ref[...], axis=0)


def _tc_reduce2(pa, pb, rows, cols):
    blk = 8192 if cols % 8192 == 0 else 2048
    return pl.pallas_call(
        _red2_body,
        out_shape=(jax.ShapeDtypeStruct((cols,), jnp.float32),
                   jax.ShapeDtypeStruct((cols,), jnp.float32)),
        grid=(cols // blk,),
        in_specs=[pl.BlockSpec((rows, blk), lambda i: (0, i)),
                  pl.BlockSpec((rows, blk), lambda i: (0, i))],
        out_specs=(pl.BlockSpec((blk,), lambda i: (i,)),
                   pl.BlockSpec((blk,), lambda i: (i,))),
    )(pa.reshape(rows, cols), pb.reshape(rows, cols))


def _h_body(num_ref, den_ref, bias_ref, xlin_ref, w_ref, b_ref, o_ref):
    conv = num_ref[...] / (den_ref[...] + jnp.float32(1e-16)) + bias_ref[...]
    h = jnp.maximum(conv + xlin_ref[...], 0.0)
    o_ref[...] = (jnp.dot(h, w_ref[...],
                          preferred_element_type=jnp.float32) + b_ref[...])


def _tc_combine(num_t, den_t, bias, xlin, w, b):
    n, c = num_t.shape
    k = w.shape[1]
    blk = 2048
    return pl.pallas_call(
        _h_body,
        out_shape=jax.ShapeDtypeStruct((n, k), jnp.float32),
        grid=(n // blk,),
        in_specs=[pl.BlockSpec((blk, c), lambda i: (i, 0)),
                  pl.BlockSpec((blk, c), lambda i: (i, 0)),
                  pl.BlockSpec((1, c), lambda i: (0, 0)),
                  pl.BlockSpec((blk, c), lambda i: (i, 0)),
                  pl.BlockSpec((c, k), lambda i: (0, 0)),
                  pl.BlockSpec((1, k), lambda i: (0, 0))],
        out_specs=pl.BlockSpec((blk, k), lambda i: (i, 0)),
    )(num_t, den_t, bias.reshape(1, c), xlin, w, b.reshape(1, k))


def _fin_body(num_ref, den_ref, bias_ref, xlin_ref, w3_ref, b3_ref,
              w4_ref, b4_ref, w5_ref, b5_ref, wo_ref, bo_ref, o_ref):
    conv = num_ref[...] / (den_ref[...] + jnp.float32(1e-16)) + bias_ref[...]
    g = jnp.maximum(conv + xlin_ref[...], 0.0)
    g = jnp.maximum(jnp.dot(g, w3_ref[...],
                            preferred_element_type=jnp.float32) + b3_ref[...],
                    0.0)
    g = jnp.maximum(jnp.dot(g, w4_ref[...],
                            preferred_element_type=jnp.float32) + b4_ref[...],
                    0.0)
    g = jnp.maximum(g * w5_ref[0, 0] + b5_ref[...], 0.0)
    o = g * wo_ref[0, 0] + bo_ref[...]
    o_ref[...] = jax.nn.log_sigmoid(o)


def _tc_final(num_t, den_t, bias, xlin, W3, b3, W4, b4, W5, b5, Wo, bo):
    n, c = num_t.shape
    blk = 2048
    small = [(W3, (c, c)), (b3, (1, c)), (W4, (c, 1)), (b4, (1, 1)),
             (W5, (1, 1)), (b5, (1, 1)), (Wo, (1, 1)), (bo, (1, 1))]
    return pl.pallas_call(
        _fin_body,
        out_shape=jax.ShapeDtypeStruct((n, 1), jnp.float32),
        grid=(n // blk,),
        in_specs=[pl.BlockSpec((blk, c), lambda i: (i, 0)),
                  pl.BlockSpec((blk, c), lambda i: (i, 0)),
                  pl.BlockSpec((1, c), lambda i: (0, 0)),
                  pl.BlockSpec((blk, c), lambda i: (i, 0))] + [
                  pl.BlockSpec(s, lambda i: (0, 0)) for _, s in small],
        out_specs=pl.BlockSpec((blk, 1), lambda i: (i, 0)),
    )(num_t, den_t, bias.reshape(1, c), xlin,
      *[a.reshape(s) for a, s in small])


def _eighth_major(a, n8):
    return a.T.reshape(-1)


def _node_major(flat, n8):
    return flat.reshape(2 * n8, NP).T


def _edge_phase(xl, xr, src_p, dst_p, att, t, n8, sc_p1, sc_p2):
    xl8 = _eighth_major(xl, n8)
    xr8 = _eighth_major(xr, n8)
    attb = jnp.repeat(att.astype(jnp.float32), L)
    attb = jnp.pad(attb, (0, 16 * L - attb.shape[0]))
    p, s1_parts = sc_p1(xl8, xr8, src_p, dst_p, attb)
    s1_tot = _tc_reduce(s1_parts, NW, NP)
    tb = jnp.full((L,), t, jnp.float32)
    num_parts, den_parts = sc_p2(xl8, src_p, dst_p, p, s1_tot, tb)
    num, den = _tc_reduce2(num_parts, den_parts, NW, n8 * NP * 2)
    return _node_major(num, n8), _node_major(den, n8)


def kernel(x, edge_index, batch, Wl1, bl1, Wr1, br1, att1, bias1, t1,
           W_lin1, b_lin1, Wl2, bl2, Wr2, br2, att2, bias2, t2, W_lin2,
           b_lin2, W3, b3, W4, b4, W5, b5, Wo, bo):
    x_p = jnp.pad(x, ((0, NP - N), (0, 0)))
    src_p = jnp.concatenate(
        [edge_index[0], jnp.zeros((EP - E,), edge_index.dtype)]
    ).astype(jnp.int32)
    dst_p = jnp.concatenate(
        [edge_index[1], jnp.full((EP - E,), N, edge_index.dtype)]
    ).astype(jnp.int32)

    wcat1 = jnp.concatenate([Wl1, Wr1, W_lin1], axis=1)
    bcat1 = jnp.concatenate([bl1, br1, b_lin1], axis=0)
    lrs1 = _tc_matmul(x_p, wcat1, bcat1)
    xl1, xr1, xlin1 = lrs1[:, :16], lrs1[:, 16:32], lrs1[:, 32:48]
    num1, den1 = _edge_phase(xl1, xr1, src_p, dst_p, att1, t1,
                             8, _SC_P1_16, _SC_P2_16)

    wcat2 = jnp.concatenate([Wl2, Wr2, W_lin2], axis=1)
    bcat2 = jnp.concatenate([bl2, br2, b_lin2], axis=0)
    lrs2 = _tc_combine(num1, den1, bias1, xlin1, wcat2, bcat2)
    xl2, xr2, xlin2 = lrs2[:, :8], lrs2[:, 8:16], lrs2[:, 16:24]
    num2, den2 = _edge_phase(xl2, xr2, src_p, dst_p, att2, t2,
                             4, _SC_P1_8, _SC_P2_8)

    out = _tc_final(num2, den2, bias2, xlin2, W3, b3, W4, b4, W5, b5, Wo, bo)
    return out[:N]


# speedup vs baseline: 2.1840x; 2.1840x over previous
import jax
import jax.numpy as jnp
from jax import lax
from jax.experimental import pallas as pl
from jax.experimental.pallas import tpu as pltpu
from jax.experimental.pallas import tpu_sc as plsc

N = 10000
NP = 10240
E = 320000
NC, NS, L = 2, 16, 16
NW = NC * NS
EW = E // NW
NBLK = EW // L
BLK = 2048

_MESH = plsc.VectorSubcoreMesh(
    core_axis_name="c", subcore_axis_name="s", num_cores=NC, num_subcores=NS)
_SC_PARAMS = pltpu.CompilerParams(needs_layout_passes=False)


def _wid():
    return lax.axis_index("s") * NC + lax.axis_index("c")


def _make_sc_pass1(n8):
    def body(xl8_hbm, xr8_hbm, src_hbm, dst_hbm, attb_hbm,
             p_hbm, s1_hbm,
             src_v, dst_v, lg_v, s1_v, attb_v, xl_v0, xl_v1, xr_v0, xr_v1,
             sem_l, sem_r):
        w = _wid()
        base = w * EW
        xlb = (xl_v0, xl_v1)
        xrb = (xr_v0, xr_v1)
        cpl = pltpu.async_copy(xl8_hbm.at[pl.ds(0, NP * 2)], xl_v0, sem_l)
        cpr = pltpu.async_copy(xr8_hbm.at[pl.ds(0, NP * 2)], xr_v0, sem_r)
        pltpu.sync_copy(src_hbm.at[pl.ds(base, EW)], src_v)
        pltpu.sync_copy(dst_hbm.at[pl.ds(base, EW)], dst_v)
        pltpu.sync_copy(attb_hbm, attb_v)

        @plsc.parallel_loop(0, NBLK, unroll=4)
        def _(b):
            lg_v[pl.ds(b * L, L)] = jnp.zeros((L,), jnp.float32)

        @plsc.parallel_loop(0, NP // L, unroll=4)
        def _(b):
            s1_v[pl.ds(b * L, L)] = jnp.zeros((L,), jnp.float32)

        for e in range(n8):
            xl_v = xlb[e % 2]
            xr_v = xrb[e % 2]
            cpl.wait()
            cpr.wait()
            if e + 1 < n8:
                cpl = pltpu.async_copy(
                    xl8_hbm.at[pl.ds((e + 1) * NP * 2, NP * 2)],
                    xlb[(e + 1) % 2], sem_l)
                cpr = pltpu.async_copy(
                    xr8_hbm.at[pl.ds((e + 1) * NP * 2, NP * 2)],
                    xrb[(e + 1) % 2], sem_r)
            att0 = attb_v[pl.ds((e * 2 + 0) * L, L)]
            att1 = attb_v[pl.ds((e * 2 + 1) * L, L)]

            @plsc.parallel_loop(0, NBLK, unroll=4)
            def _(b, att0=att0, att1=att1):
                s16 = src_v[pl.ds(b * L, L)]
                d16 = dst_v[pl.ds(b * L, L)]
                acc = lg_v[pl.ds(b * L, L)]
                z0 = (plsc.load_gather(xl_v, [s16]) +
                      plsc.load_gather(xr_v, [d16]))
                z0 = jnp.maximum(z0, 0.2 * z0)
                acc = acc + z0 * att0
                z1 = (plsc.load_gather(xl_v, [s16 + NP]) +
                      plsc.load_gather(xr_v, [d16 + NP]))
                z1 = jnp.maximum(z1, 0.2 * z1)
                acc = acc + z1 * att1
                lg_v[pl.ds(b * L, L)] = acc

        @plsc.parallel_loop(0, NBLK, unroll=4)
        def _(b):
            p16 = jnp.exp(lg_v[pl.ds(b * L, L)])
            lg_v[pl.ds(b * L, L)] = p16
            d16 = dst_v[pl.ds(b * L, L)]
            plsc.addupdate_scatter(s1_v, [d16], p16)

        pltpu.sync_copy(lg_v, p_hbm.at[pl.ds(base, EW)])
        pltpu.sync_copy(s1_v, s1_hbm.at[pl.ds(w * NP, NP)])

    return pl.kernel(
        body,
        out_type=(jax.ShapeDtypeStruct((E,), jnp.float32),
                  jax.ShapeDtypeStruct((NW * NP,), jnp.float32)),
        mesh=_MESH,
        compiler_params=_SC_PARAMS,
        scratch_types=[
            pltpu.VMEM((EW,), jnp.int32),
            pltpu.VMEM((EW,), jnp.int32),
            pltpu.VMEM((EW,), jnp.float32),
            pltpu.VMEM((NP,), jnp.float32),
            pltpu.VMEM((16 * L,), jnp.float32),
            pltpu.VMEM((NP * 2,), jnp.float32),
            pltpu.VMEM((NP * 2,), jnp.float32),
            pltpu.VMEM((NP * 2,), jnp.float32),
            pltpu.VMEM((NP * 2,), jnp.float32),
            pltpu.SemaphoreType.DMA,
            pltpu.SemaphoreType.DMA,
        ],
    )


def _make_sc_pass2(n8):
    ow = NP * 2

    def body(xl8_hbm, src_hbm, dst_hbm, p_hbm, s1t_hbm, tb_hbm,
             num_hbm, den_hbm,
             src_v, dst_v, al_v, s1t_v, tb_v, xl_v0, xl_v1, num_v, den_v,
             sem_l):
        w = _wid()
        base = w * EW
        xlb = (xl_v0, xl_v1)
        cpl = pltpu.async_copy(xl8_hbm.at[pl.ds(0, NP * 2)], xl_v0, sem_l)
        pltpu.sync_copy(src_hbm.at[pl.ds(base, EW)], src_v)
        pltpu.sync_copy(dst_hbm.at[pl.ds(base, EW)], dst_v)
        pltpu.sync_copy(p_hbm.at[pl.ds(base, EW)], al_v)
        pltpu.sync_copy(s1t_hbm, s1t_v)
        pltpu.sync_copy(tb_hbm, tb_v)
        tv = tb_v[...]

        @plsc.parallel_loop(0, NBLK, unroll=4)
        def _(b):
            d16 = dst_v[pl.ds(b * L, L)]
            sg = plsc.load_gather(s1t_v, [d16])
            al_v[pl.ds(b * L, L)] = (al_v[pl.ds(b * L, L)] /
                                     (sg + jnp.float32(1e-16)))

        for e in range(n8):
            xl_v = xlb[e % 2]
            cpl.wait()
            if e + 1 < n8:
                cpl = pltpu.async_copy(
                    xl8_hbm.at[pl.ds((e + 1) * NP * 2, NP * 2)],
                    xlb[(e + 1) % 2], sem_l)

            @plsc.parallel_loop(0, ow // L, unroll=4)
            def _(b):
                num_v[pl.ds(b * L, L)] = jnp.zeros((L,), jnp.float32)
                den_v[pl.ds(b * L, L)] = jnp.zeros((L,), jnp.float32)

            @plsc.parallel_loop(0, NBLK, unroll=4)
            def _(b):
                s16 = src_v[pl.ds(b * L, L)]
                d16 = dst_v[pl.ds(b * L, L)]
                a16 = al_v[pl.ds(b * L, L)]
                m0 = plsc.load_gather(xl_v, [s16]) * a16
                q0 = jnp.exp(m0 * tv)
                plsc.addupdate_scatter(den_v, [d16], q0)
                plsc.addupdate_scatter(num_v, [d16], q0 * m0)
                m1 = plsc.load_gather(xl_v, [s16 + NP]) * a16
                q1 = jnp.exp(m1 * tv)
                plsc.addupdate_scatter(den_v, [d16 + NP], q1)
                plsc.addupdate_scatter(num_v, [d16 + NP], q1 * m1)

            off = (w * n8 + e) * ow
            pltpu.sync_copy(num_v, num_hbm.at[pl.ds(off, ow)])
            pltpu.sync_copy(den_v, den_hbm.at[pl.ds(off, ow)])

    return pl.kernel(
        body,
        out_type=(jax.ShapeDtypeStruct((NW * n8 * ow,), jnp.float32),
                  jax.ShapeDtypeStruct((NW * n8 * ow,), jnp.float32)),
        mesh=_MESH,
        compiler_params=_SC_PARAMS,
        scratch_types=[
            pltpu.VMEM((EW,), jnp.int32),
            pltpu.VMEM((EW,), jnp.int32),
            pltpu.VMEM((EW,), jnp.float32),
            pltpu.VMEM((NP,), jnp.float32),
            pltpu.VMEM((L,), jnp.float32),
            pltpu.VMEM((NP * 2,), jnp.float32),
            pltpu.VMEM((NP * 2,), jnp.float32),
            pltpu.VMEM((NP * 2,), jnp.float32),
            pltpu.VMEM((NP * 2,), jnp.float32),
            pltpu.SemaphoreType.DMA,
        ],
    )


_SC_P1_16 = _make_sc_pass1(8)
_SC_P2_16 = _make_sc_pass2(8)
_SC_P1_8 = _make_sc_pass1(4)
_SC_P2_8 = _make_sc_pass2(4)


def _mm_body(x_ref, w_ref, b_ref, xlT_ref, xrT_ref, xlin_ref):
    m = (jnp.dot(x_ref[...], w_ref[...],
                 preferred_element_type=jnp.float32) + b_ref[...])
    xlT_ref[...] = m[:, :16].T
    xrT_ref[...] = m[:, 16:32].T
    xlin_ref[...] = m[:, 32:48]


def _tc_matmul(x, w, b):
    return pl.pallas_call(
        _mm_body,
        out_shape=(jax.ShapeDtypeStruct((16, NP), jnp.float32),
                   jax.ShapeDtypeStruct((16, NP), jnp.float32),
                   jax.ShapeDtypeStruct((NP, 16), jnp.float32)),
        grid=(NP // BLK,),
        in_specs=[pl.BlockSpec((BLK, 128), lambda i: (i, 0)),
                  pl.BlockSpec((128, 48), lambda i: (0, 0)),
                  pl.BlockSpec((1, 48), lambda i: (0, 0))],
        out_specs=(pl.BlockSpec((16, BLK), lambda i: (0, i)),
                   pl.BlockSpec((16, BLK), lambda i: (0, i)),
                   pl.BlockSpec((BLK, 16), lambda i: (i, 0))),
    )(x, w, b.reshape(1, 48))


def _red_body(p_ref, o_ref):
    o_ref[...] = jnp.sum(p_ref[...], axis=0)


def _tc_reduce_s1(parts):
    return pl.pallas_call(
        _red_body,
        out_shape=jax.ShapeDtypeStruct((NP,), jnp.float32),
        grid=(NP // BLK,),
        in_specs=[pl.BlockSpec((NW, BLK), lambda i: (0, i))],
        out_specs=pl.BlockSpec((BLK,), lambda i: (i,)),
    )(parts.reshape(NW, NP))


def _comb_body(nump_ref, denp_ref, bias_ref, xlin_ref, w_ref, b_ref,
               xlT_ref, xrT_ref, xlin2_ref):
    num = jnp.sum(nump_ref[...], axis=0).T
    den = jnp.sum(denp_ref[...], axis=0).T
    conv = num / (den + jnp.float32(1e-16)) + bias_ref[...]
    h = jnp.maximum(conv + xlin_ref[...], 0.0)
    m = (jnp.dot(h, w_ref[...],
                 preferred_element_type=jnp.float32) + b_ref[...])
    xlT_ref[...] = m[:, :8].T
    xrT_ref[...] = m[:, 8:16].T
    xlin2_ref[...] = m[:, 16:24]


def _tc_combine(num_parts, den_parts, bias, xlin, w, b):
    return pl.pallas_call(
        _comb_body,
        out_shape=(jax.ShapeDtypeStruct((8, NP), jnp.float32),
                   jax.ShapeDtypeStruct((8, NP), jnp.float32),
                   jax.ShapeDtypeStruct((NP, 8), jnp.float32)),
        grid=(NP // BLK,),
        in_specs=[pl.BlockSpec((NW, 16, BLK), lambda i: (0, 0, i)),
                  pl.BlockSpec((NW, 16, BLK), lambda i: (0, 0, i)),
                  pl.BlockSpec((1, 16), lambda i: (0, 0)),
                  pl.BlockSpec((BLK, 16), lambda i: (i, 0)),
                  pl.BlockSpec((16, 24), lambda i: (0, 0)),
                  pl.BlockSpec((1, 24), lambda i: (0, 0))],
        out_specs=(pl.BlockSpec((8, BLK), lambda i: (0, i)),
                   pl.BlockSpec((8, BLK), lambda i: (0, i)),
                   pl.BlockSpec((BLK, 8), lambda i: (i, 0))),
    )(num_parts.reshape(NW, 16, NP), den_parts.reshape(NW, 16, NP),
      bias.reshape(1, 16), xlin, w, b.reshape(1, 24))


def _fin_body(nump_ref, denp_ref, bias_ref, xlin_ref, w3_ref, b3_ref,
              w4_ref, b4_ref, w5_ref, b5_ref, wo_ref, bo_ref, o_ref):
    num = jnp.sum(nump_ref[...], axis=0).T
    den = jnp.sum(denp_ref[...], axis=0).T
    conv = num / (den + jnp.float32(1e-16)) + bias_ref[...]
    g = jnp.maximum(conv + xlin_ref[...], 0.0)
    g = jnp.maximum(jnp.dot(g, w3_ref[...],
                            preferred_element_type=jnp.float32) + b3_ref[...],
                    0.0)
    g = jnp.maximum(jnp.dot(g, w4_ref[...],
                            preferred_element_type=jnp.float32) + b4_ref[...],
                    0.0)
    g = jnp.maximum(g * w5_ref[0, 0] + b5_ref[...], 0.0)
    o = g * wo_ref[0, 0] + bo_ref[...]
    o_ref[...] = jax.nn.log_sigmoid(o)


def _tc_final(num_parts, den_parts, bias, xlin, W3, b3, W4, b4, W5, b5,
              Wo, bo):
    small = [(W3, (8, 8)), (b3, (1, 8)), (W4, (8, 1)), (b4, (1, 1)),
             (W5, (1, 1)), (b5, (1, 1)), (Wo, (1, 1)), (bo, (1, 1))]
    return pl.pallas_call(
        _fin_body,
        out_shape=jax.ShapeDtypeStruct((NP, 1), jnp.float32),
        grid=(NP // BLK,),
        in_specs=[pl.BlockSpec((NW, 8, BLK), lambda i: (0, 0, i)),
                  pl.BlockSpec((NW, 8, BLK), lambda i: (0, 0, i)),
                  pl.BlockSpec((1, 8), lambda i: (0, 0)),
                  pl.BlockSpec((BLK, 8), lambda i: (i, 0))] + [
                  pl.BlockSpec(s, lambda i: (0, 0)) for _, s in small],
        out_specs=pl.BlockSpec((BLK, 1), lambda i: (i, 0)),
    )(num_parts.reshape(NW, 8, NP), den_parts.reshape(NW, 8, NP),
      bias.reshape(1, 8), xlin, *[a.reshape(s) for a, s in small])


def _edge_phase(xlT, xrT, src, dst, att, t, n8, sc_p1, sc_p2):
    attb = jnp.repeat(att.astype(jnp.float32), L)
    attb = jnp.pad(attb, (0, 16 * L - attb.shape[0]))
    p, s1_parts = sc_p1(xlT.reshape(-1), xrT.reshape(-1), src, dst, attb)
    s1_tot = _tc_reduce_s1(s1_parts)
    tb = jnp.full((L,), t, jnp.float32)
    return sc_p2(xlT.reshape(-1), src, dst, p, s1_tot, tb)


def kernel(x, edge_index, batch, Wl1, bl1, Wr1, br1, att1, bias1, t1,
           W_lin1, b_lin1, Wl2, bl2, Wr2, br2, att2, bias2, t2, W_lin2,
           b_lin2, W3, b3, W4, b4, W5, b5, Wo, bo):
    src = edge_index[0].astype(jnp.int32)
    dst = edge_index[1].astype(jnp.int32)

    wcat1 = jnp.concatenate([Wl1, Wr1, W_lin1], axis=1)
    bcat1 = jnp.concatenate([bl1, br1, b_lin1], axis=0)
    x_p = jnp.pad(x, ((0, NP - N), (0, 0)))
    xl1T, xr1T, xlin1 = _tc_matmul(x_p, wcat1, bcat1)
    num1p, den1p = _edge_phase(xl1T, xr1T, src, dst, att1, t1,
                               8, _SC_P1_16, _SC_P2_16)

    wcat2 = jnp.concatenate([Wl2, Wr2, W_lin2], axis=1)
    bcat2 = jnp.concatenate([bl2, br2, b_lin2], axis=0)
    xl2T, xr2T, xlin2 = _tc_combine(num1p, den1p, bias1, xlin1,
                                    wcat2, bcat2)
    num2p, den2p = _edge_phase(xl2T, xr2T, src, dst, att2, t2,
                               4, _SC_P1_8, _SC_P2_8)

    out = _tc_final(num2p, den2p, bias2, xlin2,
                    W3, b3, W4, b4, W5, b5, Wo, bo)
    return out[:N]


# speedup vs baseline: 2.1921x; 1.0037x over previous
import jax
import jax.numpy as jnp
from jax import lax
from jax.experimental import pallas as pl
from jax.experimental.pallas import tpu as pltpu
from jax.experimental.pallas import tpu_sc as plsc

N = 10000
NP = 10240
E = 320000
NC, NS, L = 2, 16, 16
NW = NC * NS
EW = E // NW
NBLK = EW // L
BLK = 2048

_MESH = plsc.VectorSubcoreMesh(
    core_axis_name="c", subcore_axis_name="s", num_cores=NC, num_subcores=NS)
_SC_PARAMS = pltpu.CompilerParams(needs_layout_passes=False)


def _wid():
    return lax.axis_index("s") * NC + lax.axis_index("c")


def _make_sc_pass1(n8):
    def body(xl8_hbm, xr8_hbm, src_hbm, dst_hbm, attb_hbm,
             p_hbm, s1_hbm,
             src_v, dst_v, lg_v, s1_v, attb_v, xl_v0, xl_v1, xr_v0, xr_v1,
             sem_l, sem_r):
        w = _wid()
        base = w * EW
        xlb = (xl_v0, xl_v1)
        xrb = (xr_v0, xr_v1)
        cpl = pltpu.async_copy(xl8_hbm.at[pl.ds(0, NP * 2)], xl_v0, sem_l)
        cpr = pltpu.async_copy(xr8_hbm.at[pl.ds(0, NP * 2)], xr_v0, sem_r)
        pltpu.sync_copy(src_hbm.at[pl.ds(base, EW)], src_v)
        pltpu.sync_copy(dst_hbm.at[pl.ds(base, EW)], dst_v)
        pltpu.sync_copy(attb_hbm, attb_v)

        @plsc.parallel_loop(0, NBLK, unroll=4)
        def _(b):
            lg_v[pl.ds(b * L, L)] = jnp.zeros((L,), jnp.float32)

        @plsc.parallel_loop(0, NP // L, unroll=4)
        def _(b):
            s1_v[pl.ds(b * L, L)] = jnp.zeros((L,), jnp.float32)

        for e in range(n8):
            xl_v = xlb[e % 2]
            xr_v = xrb[e % 2]
            cpl.wait()
            cpr.wait()
            if e + 1 < n8:
                cpl = pltpu.async_copy(
                    xl8_hbm.at[pl.ds((e + 1) * NP * 2, NP * 2)],
                    xlb[(e + 1) % 2], sem_l)
                cpr = pltpu.async_copy(
                    xr8_hbm.at[pl.ds((e + 1) * NP * 2, NP * 2)],
                    xrb[(e + 1) % 2], sem_r)
            att0 = attb_v[pl.ds((e * 2 + 0) * L, L)]
            att1 = attb_v[pl.ds((e * 2 + 1) * L, L)]

            @plsc.parallel_loop(0, NBLK, unroll=4)
            def _(b, att0=att0, att1=att1):
                s16 = src_v[pl.ds(b * L, L)]
                d16 = dst_v[pl.ds(b * L, L)]
                acc = lg_v[pl.ds(b * L, L)]
                z0 = (plsc.load_gather(xl_v, [s16]) +
                      plsc.load_gather(xr_v, [d16]))
                z0 = jnp.maximum(z0, 0.2 * z0)
                acc = acc + z0 * att0
                z1 = (plsc.load_gather(xl_v, [s16 + NP]) +
                      plsc.load_gather(xr_v, [d16 + NP]))
                z1 = jnp.maximum(z1, 0.2 * z1)
                acc = acc + z1 * att1
                lg_v[pl.ds(b * L, L)] = acc

        @plsc.parallel_loop(0, NBLK, unroll=4)
        def _(b):
            p16 = jnp.exp(lg_v[pl.ds(b * L, L)])
            lg_v[pl.ds(b * L, L)] = p16
            d16 = dst_v[pl.ds(b * L, L)]
            plsc.addupdate_scatter(s1_v, [d16], p16)

        pltpu.sync_copy(lg_v, p_hbm.at[pl.ds(base, EW)])
        pltpu.sync_copy(s1_v, s1_hbm.at[pl.ds(w * NP, NP)])

    return pl.kernel(
        body,
        out_type=(jax.ShapeDtypeStruct((E,), jnp.float32),
                  jax.ShapeDtypeStruct((NW * NP,), jnp.float32)),
        mesh=_MESH,
        compiler_params=_SC_PARAMS,
        scratch_types=[
            pltpu.VMEM((EW,), jnp.int32),
            pltpu.VMEM((EW,), jnp.int32),
            pltpu.VMEM((EW,), jnp.float32),
            pltpu.VMEM((NP,), jnp.float32),
            pltpu.VMEM((16 * L,), jnp.float32),
            pltpu.VMEM((NP * 2,), jnp.float32),
            pltpu.VMEM((NP * 2,), jnp.float32),
            pltpu.VMEM((NP * 2,), jnp.float32),
            pltpu.VMEM((NP * 2,), jnp.float32),
            pltpu.SemaphoreType.DMA,
            pltpu.SemaphoreType.DMA,
        ],
    )


def _make_sc_pass2(n8):
    ow = NP * 2

    def body(xl8_hbm, src_hbm, dst_hbm, p_hbm, s1t_hbm, tb_hbm,
             num_hbm, den_hbm,
             src_v, dst_v, al_v, s1t_v, tb_v, xl_v0, xl_v1, num_v, den_v,
             sem_l):
        w = _wid()
        base = w * EW
        xlb = (xl_v0, xl_v1)
        cpl = pltpu.async_copy(xl8_hbm.at[pl.ds(0, NP * 2)], xl_v0, sem_l)
        pltpu.sync_copy(src_hbm.at[pl.ds(base, EW)], src_v)
        pltpu.sync_copy(dst_hbm.at[pl.ds(base, EW)], dst_v)
        pltpu.sync_copy(p_hbm.at[pl.ds(base, EW)], al_v)
        pltpu.sync_copy(s1t_hbm, s1t_v)
        pltpu.sync_copy(tb_hbm, tb_v)
        tv = tb_v[...]

        @plsc.parallel_loop(0, NBLK, unroll=4)
        def _(b):
            d16 = dst_v[pl.ds(b * L, L)]
            sg = plsc.load_gather(s1t_v, [d16])
            al_v[pl.ds(b * L, L)] = (al_v[pl.ds(b * L, L)] /
                                     (sg + jnp.float32(1e-16)))

        for e in range(n8):
            xl_v = xlb[e % 2]
            cpl.wait()
            if e + 1 < n8:
                cpl = pltpu.async_copy(
                    xl8_hbm.at[pl.ds((e + 1) * NP * 2, NP * 2)],
                    xlb[(e + 1) % 2], sem_l)

            @plsc.parallel_loop(0, ow // L, unroll=4)
            def _(b):
                num_v[pl.ds(b * L, L)] = jnp.zeros((L,), jnp.float32)
                den_v[pl.ds(b * L, L)] = jnp.zeros((L,), jnp.float32)

            @plsc.parallel_loop(0, NBLK, unroll=4)
            def _(b):
                s16 = src_v[pl.ds(b * L, L)]
                d16 = dst_v[pl.ds(b * L, L)]
                a16 = al_v[pl.ds(b * L, L)]
                m0 = plsc.load_gather(xl_v, [s16]) * a16
                q0 = jnp.exp(m0 * tv)
                plsc.addupdate_scatter(den_v, [d16], q0)
                plsc.addupdate_scatter(num_v, [d16], q0 * m0)
                m1 = plsc.load_gather(xl_v, [s16 + NP]) * a16
                q1 = jnp.exp(m1 * tv)
                plsc.addupdate_scatter(den_v, [d16 + NP], q1)
                plsc.addupdate_scatter(num_v, [d16 + NP], q1 * m1)

            off = (w * n8 + e) * ow
            pltpu.sync_copy(num_v, num_hbm.at[pl.ds(off, ow)])
            pltpu.sync_copy(den_v, den_hbm.at[pl.ds(off, ow)])

    return pl.kernel(
        body,
        out_type=(jax.ShapeDtypeStruct((NW * n8 * ow,), jnp.float32),
                  jax.ShapeDtypeStruct((NW * n8 * ow,), jnp.float32)),
        mesh=_MESH,
        compiler_params=_SC_PARAMS,
        scratch_types=[
            pltpu.VMEM((EW,), jnp.int32),
            pltpu.VMEM((EW,), jnp.int32),
            pltpu.VMEM((EW,), jnp.float32),
            pltpu.VMEM((NP,), jnp.float32),
            pltpu.VMEM((L,), jnp.float32),
            pltpu.VMEM((NP * 2,), jnp.float32),
            pltpu.VMEM((NP * 2,), jnp.float32),
            pltpu.VMEM((NP * 2,), jnp.float32),
            pltpu.VMEM((NP * 2,), jnp.float32),
            pltpu.SemaphoreType.DMA,
        ],
    )


_SC_P1_16 = _make_sc_pass1(8)
_SC_P2_16 = _make_sc_pass2(8)
_SC_P1_8 = _make_sc_pass1(4)
_SC_P2_8 = _make_sc_pass2(4)


def _mm_body(x_ref, wl_ref, wr_ref, wn_ref, bl_ref, br_ref, bn_ref,
             a1_ref, t1_ref, a2_ref, t2_ref,
             xlT_ref, xrT_ref, xlin_ref, ab1_ref, tb1_ref, ab2_ref,
             tb2_ref):
    x = x_ref[...]
    xlT_ref[...] = (jnp.dot(x, wl_ref[...],
                            preferred_element_type=jnp.float32)
                    + bl_ref[...]).T
    xrT_ref[...] = (jnp.dot(x, wr_ref[...],
                            preferred_element_type=jnp.float32)
                    + br_ref[...]).T
    xlin_ref[...] = (jnp.dot(x, wn_ref[...],
                             preferred_element_type=jnp.float32)
                     + bn_ref[...])

    @pl.when(pl.program_id(0) == 0)
    def _():
        ab1_ref[...] = jnp.broadcast_to(a1_ref[...].T, (16, L))
        tb1_ref[...] = jnp.broadcast_to(t1_ref[...], (1, L))
        a2p = jnp.pad(a2_ref[...], ((0, 0), (0, 8)))
        ab2_ref[...] = jnp.broadcast_to(a2p.T, (16, L))
        tb2_ref[...] = jnp.broadcast_to(t2_ref[...], (1, L))


def _tc_matmul(x, wl, wr, wn, bl, br, bn, a1, t1, a2, t2):
    return pl.pallas_call(
        _mm_body,
        out_shape=(jax.ShapeDtypeStruct((16, NP), jnp.float32),
                   jax.ShapeDtypeStruct((16, NP), jnp.float32),
                   jax.ShapeDtypeStruct((NP, 16), jnp.float32),
                   jax.ShapeDtypeStruct((16, L), jnp.float32),
                   jax.ShapeDtypeStruct((1, L), jnp.float32),
                   jax.ShapeDtypeStruct((16, L), jnp.float32),
                   jax.ShapeDtypeStruct((1, L), jnp.float32)),
        grid=(NP // BLK,),
        in_specs=[pl.BlockSpec((BLK, 128), lambda i: (i, 0)),
                  pl.BlockSpec((128, 16), lambda i: (0, 0)),
                  pl.BlockSpec((128, 16), lambda i: (0, 0)),
                  pl.BlockSpec((128, 16), lambda i: (0, 0)),
                  pl.BlockSpec((1, 16), lambda i: (0, 0)),
                  pl.BlockSpec((1, 16), lambda i: (0, 0)),
                  pl.BlockSpec((1, 16), lambda i: (0, 0)),
                  pl.BlockSpec((1, 16), lambda i: (0, 0)),
                  pl.BlockSpec((1, 1), lambda i: (0, 0)),
                  pl.BlockSpec((1, 8), lambda i: (0, 0)),
                  pl.BlockSpec((1, 1), lambda i: (0, 0))],
        out_specs=(pl.BlockSpec((16, BLK), lambda i: (0, i)),
                   pl.BlockSpec((16, BLK), lambda i: (0, i)),
                   pl.BlockSpec((BLK, 16), lambda i: (i, 0)),
                   pl.BlockSpec((16, L), lambda i: (0, 0)),
                   pl.BlockSpec((1, L), lambda i: (0, 0)),
                   pl.BlockSpec((16, L), lambda i: (0, 0)),
                   pl.BlockSpec((1, L), lambda i: (0, 0))),
    )(x, wl, wr, wn, bl.reshape(1, 16), br.reshape(1, 16),
      bn.reshape(1, 16), a1.reshape(1, 16), t1.reshape(1, 1),
      a2.reshape(1, 8), t2.reshape(1, 1))


def _red_body(p_ref, o_ref):
    o_ref[...] = jnp.sum(p_ref[...], axis=0)


def _tc_reduce_s1(parts):
    return pl.pallas_call(
        _red_body,
        out_shape=jax.ShapeDtypeStruct((NP,), jnp.float32),
        grid=(NP // BLK,),
        in_specs=[pl.BlockSpec((NW, BLK), lambda i: (0, i))],
        out_specs=pl.BlockSpec((BLK,), lambda i: (i,)),
    )(parts.reshape(NW, NP))


def _comb_body(nump_ref, denp_ref, bias_ref, xlin_ref, wl_ref, wr_ref,
               wn_ref, bl_ref, br_ref, bn_ref,
               xlT_ref, xrT_ref, xlin2_ref):
    num = jnp.sum(nump_ref[...], axis=0).T
    den = jnp.sum(denp_ref[...], axis=0).T
    conv = num / (den + jnp.float32(1e-16)) + bias_ref[...]
    h = jnp.maximum(conv + xlin_ref[...], 0.0)
    xlT_ref[...] = (jnp.dot(h, wl_ref[...],
                            preferred_element_type=jnp.float32)
                    + bl_ref[...]).T
    xrT_ref[...] = (jnp.dot(h, wr_ref[...],
                            preferred_element_type=jnp.float32)
                    + br_ref[...]).T
    xlin2_ref[...] = (jnp.dot(h, wn_ref[...],
                              preferred_element_type=jnp.float32)
                      + bn_ref[...])


def _tc_combine(num_parts, den_parts, bias, xlin, wl, wr, wn, bl, br, bn):
    return pl.pallas_call(
        _comb_body,
        out_shape=(jax.ShapeDtypeStruct((8, NP), jnp.float32),
                   jax.ShapeDtypeStruct((8, NP), jnp.float32),
                   jax.ShapeDtypeStruct((NP, 8), jnp.float32)),
        grid=(NP // BLK,),
        in_specs=[pl.BlockSpec((NW, 16, BLK), lambda i: (0, 0, i)),
                  pl.BlockSpec((NW, 16, BLK), lambda i: (0, 0, i)),
                  pl.BlockSpec((1, 16), lambda i: (0, 0)),
                  pl.BlockSpec((BLK, 16), lambda i: (i, 0)),
                  pl.BlockSpec((16, 8), lambda i: (0, 0)),
                  pl.BlockSpec((16, 8), lambda i: (0, 0)),
                  pl.BlockSpec((16, 8), lambda i: (0, 0)),
                  pl.BlockSpec((1, 8), lambda i: (0, 0)),
                  pl.BlockSpec((1, 8), lambda i: (0, 0)),
                  pl.BlockSpec((1, 8), lambda i: (0, 0))],
        out_specs=(pl.BlockSpec((8, BLK), lambda i: (0, i)),
                   pl.BlockSpec((8, BLK), lambda i: (0, i)),
                   pl.BlockSpec((BLK, 8), lambda i: (i, 0))),
    )(num_parts.reshape(NW, 16, NP), den_parts.reshape(NW, 16, NP),
      bias.reshape(1, 16), xlin, wl, wr, wn, bl.reshape(1, 8),
      br.reshape(1, 8), bn.reshape(1, 8))


def _fin_body(nump_ref, denp_ref, bias_ref, xlin_ref, w3_ref, b3_ref,
              w4_ref, b4_ref, w5_ref, b5_ref, wo_ref, bo_ref, o_ref):
    num = jnp.sum(nump_ref[...], axis=0).T
    den = jnp.sum(denp_ref[...], axis=0).T
    conv = num / (den + jnp.float32(1e-16)) + bias_ref[...]
    g = jnp.maximum(conv + xlin_ref[...], 0.0)
    g = jnp.maximum(jnp.dot(g, w3_ref[...],
                            preferred_element_type=jnp.float32) + b3_ref[...],
                    0.0)
    g = jnp.maximum(jnp.dot(g, w4_ref[...],
                            preferred_element_type=jnp.float32) + b4_ref[...],
                    0.0)
    g = jnp.maximum(g * w5_ref[0, 0] + b5_ref[...], 0.0)
    o = g * wo_ref[0, 0] + bo_ref[...]
    o_ref[...] = jax.nn.log_sigmoid(o)


def _tc_final(num_parts, den_parts, bias, xlin, W3, b3, W4, b4, W5, b5,
              Wo, bo):
    small = [(W3, (8, 8)), (b3, (1, 8)), (W4, (8, 1)), (b4, (1, 1)),
             (W5, (1, 1)), (b5, (1, 1)), (Wo, (1, 1)), (bo, (1, 1))]
    return pl.pallas_call(
        _fin_body,
        out_shape=jax.ShapeDtypeStruct((NP, 1), jnp.float32),
        grid=(NP // BLK,),
        in_specs=[pl.BlockSpec((NW, 8, BLK), lambda i: (0, 0, i)),
                  pl.BlockSpec((NW, 8, BLK), lambda i: (0, 0, i)),
                  pl.BlockSpec((1, 8), lambda i: (0, 0)),
                  pl.BlockSpec((BLK, 8), lambda i: (i, 0))] + [
                  pl.BlockSpec(s, lambda i: (0, 0)) for _, s in small],
        out_specs=pl.BlockSpec((BLK, 1), lambda i: (i, 0)),
    )(num_parts.reshape(NW, 8, NP), den_parts.reshape(NW, 8, NP),
      bias.reshape(1, 8), xlin, *[a.reshape(s) for a, s in small])


def _edge_phase(xlT, xrT, src, dst, attb, tb, n8, sc_p1, sc_p2):
    p, s1_parts = sc_p1(xlT.reshape(-1), xrT.reshape(-1), src, dst,
                        attb.reshape(-1))
    s1_tot = _tc_reduce_s1(s1_parts)
    return sc_p2(xlT.reshape(-1), src, dst, p, s1_tot, tb.reshape(-1))


def kernel(x, edge_index, batch, Wl1, bl1, Wr1, br1, att1, bias1, t1,
           W_lin1, b_lin1, Wl2, bl2, Wr2, br2, att2, bias2, t2, W_lin2,
           b_lin2, W3, b3, W4, b4, W5, b5, Wo, bo):
    src = edge_index[0].astype(jnp.int32)
    dst = edge_index[1].astype(jnp.int32)

    x_p = jnp.pad(x, ((0, NP - N), (0, 0)))
    (xl1T, xr1T, xlin1, attb1, tb1, attb2, tb2) = _tc_matmul(
        x_p, Wl1, Wr1, W_lin1, bl1, br1, b_lin1, att1, t1, att2, t2)
    num1p, den1p = _edge_phase(xl1T, xr1T, src, dst, attb1, tb1,
                               8, _SC_P1_16, _SC_P2_16)

    xl2T, xr2T, xlin2 = _tc_combine(num1p, den1p, bias1, xlin1,
                                    Wl2, Wr2, W_lin2, bl2, br2, b_lin2)
    num2p, den2p = _edge_phase(xl2T, xr2T, src, dst, attb2, tb2,
                               4, _SC_P1_8, _SC_P2_8)

    out = _tc_final(num2p, den2p, bias2, xlin2,
                    W3, b3, W4, b4, W5, b5, Wo, bo)
    return out[:N]
